# baseline (speedup 1.0000x reference)
import functools

import jax
import jax.numpy as jnp
from jax import lax
from jax.experimental import pallas as pl
from jax.experimental.pallas import tpu as pltpu
from jax.experimental.pallas import tpu_sc as plsc

_N = 10000
_E = 320000
_NPAD = 10240
_EPAD = 327680
_CHUNK = 128
_TILES = 32
_PER_TILE = _EPAD // _TILES
_NCHUNKS = _PER_TILE // _CHUNK
_RPT = _NPAD // 16
_PAD_DST = 10016

_R = 1000
_G = _N // _R

_f32 = jnp.float32

@functools.cache
def _sc_mesh():
    return plsc.VectorSubcoreMesh(
        core_axis_name="c", subcore_axis_name="s", num_cores=2, num_subcores=16)


_sc_params = pltpu.CompilerParams(use_tc_tiling_on_sc=False)


_NB = 4


_W = 48


_W = 48


def _agg(zs, src2d, dst2d, zeros):
    nq = len(zs)

    @functools.partial(
        pl.kernel,
        out_type=[jax.ShapeDtypeStruct((2, _NPAD, _W), _f32)] * nq,
        mesh=_sc_mesh(),
        scratch_types=[
            pltpu.VMEM((_NCHUNKS, _CHUNK), jnp.int32),
            pltpu.VMEM((_NCHUNKS, _CHUNK), jnp.int32),
            pltpu.VMEM((_NB, _CHUNK, _W), _f32),
            pltpu.VMEM_SHARED((_NPAD, _W), _f32),
            pltpu.VMEM_SHARED((_N, _W), _f32),
            pltpu.SemaphoreType.DMA,
            pltpu.SemaphoreType.DMA,
        ],
        compiler_params=_sc_params,
    )
    def k(*refs):
        z_hbms = refs[:nq]
        src_hbm, dst_hbm, zero_hbm = refs[nq:nq + 3]
        out_hbms = refs[nq + 3:2 * nq + 3]
        srci, dsti, rows, acc, zbuf, gsem, ssem = refs[2 * nq + 3:]
        cid = lax.axis_index("c")
        sid = lax.axis_index("s")
        tile = cid * 16 + sid
        rbase = sid * _RPT
        cbase = tile * _NCHUNKS
        pltpu.sync_copy(src_hbm.at[pl.ds(cbase, _NCHUNKS)], srci)
        pltpu.sync_copy(dst_hbm.at[pl.ds(cbase, _NCHUNKS)], dsti)

        for q in range(nq):
            pltpu.sync_copy(zero_hbm.at[pl.ds(rbase, _RPT)],
                            acc.at[pl.ds(rbase, _RPT)])
            pltpu.sync_copy(z_hbms[q].at[pl.ds(sid * 625, 625)],
                            zbuf.at[pl.ds(sid * 625, 625)])
            plsc.subcore_barrier()

            for b in range(_NB):
                pltpu.async_copy(zbuf.at[srci.at[b]], rows.at[b], gsem)

            @pl.loop(0, _NCHUNKS // _NB)
            def _(s):
                for b in range(_NB):
                    j = s * _NB + b
                    pltpu.make_async_copy(
                        zbuf.at[srci.at[j]], rows.at[b], gsem).wait()
                    pltpu.async_copy(rows.at[b], acc.at[dsti.at[j]], ssem,
                                     add=True)
                for b in range(_NB):
                    j = s * _NB + b
                    pltpu.make_async_copy(
                        rows.at[b], acc.at[dsti.at[j]], ssem).wait()

                    @pl.when(j + _NB < _NCHUNKS)
                    def _():
                        pltpu.async_copy(zbuf.at[srci.at[j + _NB]],
                                         rows.at[b], gsem)

            plsc.subcore_barrier()
            pltpu.sync_copy(acc.at[pl.ds(rbase, _RPT)],
                            out_hbms[q].at[cid, pl.ds(rbase, _RPT)])

    return k(*zs, src2d, dst2d, zeros)


def _sc_degree(dstp, zeros16, ones16):
    @functools.partial(
        pl.kernel,
        out_type=jax.ShapeDtypeStruct((2, _NPAD, 16), _f32),
        mesh=_sc_mesh(),
        scratch_types=[
            pltpu.VMEM((_NCHUNKS, _CHUNK), jnp.int32),
            pltpu.VMEM((_CHUNK, 16), _f32),
            pltpu.VMEM_SHARED((_NPAD, 16), _f32),
        ],
        compiler_params=_sc_params,
    )
    def k(dst_hbm, zero_hbm, ones_hbm, out_hbm, dsti, ones_v, acc):
        cid = lax.axis_index("c")
        sid = lax.axis_index("s")
        tile = cid * 16 + sid
        rbase = sid * _RPT
        cbase = tile * _NCHUNKS
        pltpu.sync_copy(zero_hbm.at[pl.ds(rbase, _RPT)], acc.at[pl.ds(rbase, _RPT)])
        pltpu.sync_copy(dst_hbm.at[pl.ds(cbase, _NCHUNKS)], dsti)
        pltpu.sync_copy(ones_hbm, ones_v)
        plsc.subcore_barrier()

        @pl.loop(0, _NCHUNKS)
        def _(j):
            pltpu.sync_copy(ones_v, acc.at[dsti.at[j]], add=True)

        plsc.subcore_barrier()
        pltpu.sync_copy(acc.at[pl.ds(rbase, _RPT)],
                        out_hbm.at[cid, pl.ds(rbase, _RPT)])

    return k(dstp, zeros16, ones16)


def _dot(a, b):
    return jnp.dot(a, b, preferred_element_type=_f32)


def _tc1_body(xm_ref, lg_ref, xt_ref,
              mW36, mcb, mpW, mpb,
              lW1, lEb1, lW2, lEb2,
              tW12, tcb, tpW, tpb,
              mWn1, mWs1, mb1, lWn1, lWs1, lb1, tWn1, tWs1, tb1,
              z1a_ref, z1b_ref, z1c_ref, z1d_ref, s1_ref):
    xm = xm_ref[...]
    accm = jnp.zeros((_R, 64), _f32)
    for t in range(18):
        accm = accm + jax.nn.relu(_dot(xm[:, 12 * t:12 * t + 36], mW36[...]) + mcb[...])
    m_emb = _dot(accm * (1.0 / 18.0), mpW[...]) + mpb[...]

    lg = lg_ref[...]
    l_emb = _dot(jax.nn.relu(_dot(lg, lW1[...]) + lEb1[...]), lW2[...]) + lEb2[...]

    xt = xt_ref[...]
    acct = jnp.zeros((_R, 64), _f32)
    for t in range(18):
        acct = acct + jax.nn.relu(_dot(xt[:, 4 * t:4 * t + 12], tW12[...]) + tcb[...])
    t_emb = _dot(acct * (1.0 / 18.0), tpW[...]) + tpb[...]

    z1 = jnp.concatenate(
        [_dot(m_emb, mWn1[...]), _dot(l_emb, lWn1[...]), _dot(t_emb, tWn1[...])],
        axis=1)
    z1a_ref[...] = z1[:, 0:48]
    z1b_ref[...] = z1[:, 48:96]
    z1c_ref[...] = z1[:, 96:144]
    z1d_ref[...] = z1[:, 144:192]
    s1_ref[...] = jnp.concatenate(
        [_dot(m_emb, mWs1[...]) + mb1[...],
         _dot(l_emb, lWs1[...]) + lb1[...],
         _dot(t_emb, tWs1[...]) + tb1[...]],
        axis=1)


def _full(shape):
    return pl.BlockSpec(shape, lambda i: tuple(0 for _ in shape))


def _rows(width):
    return pl.BlockSpec((_R, width), lambda i: (i, 0))


def _tc1(xm, lg, xt, ws):
    specs = ([_rows(240), _rows(100), _rows(80)] +
             [_full(w.shape) for w in ws])
    return pl.pallas_call(
        _tc1_body,
        grid=(_G,),
        in_specs=specs,
        out_specs=[_rows(48), _rows(48), _rows(48), _rows(48), _rows(192)],
        out_shape=[jax.ShapeDtypeStruct((_N, 48), _f32)] * 4 +
                  [jax.ShapeDtypeStruct((_N, 192), _f32)],
    )(xm, lg, xt, *ws)


def _inv_deg(deg_ref):
    d = deg_ref[...]
    dg = d[0, :, 0:1] + d[1, :, 0:1]
    return 1.0 / jnp.maximum(dg, 1.0)


def _tc2_body(s1_ref, agga_ref, aggb_ref, aggc_ref, aggd_ref, deg_ref,
              mWn2, mWs2, mb2, lWn2, lWs2, lb2, tWn2, tWs2, tb2,
              z2a_ref, z2b_ref, s2_ref):
    a = jnp.concatenate(
        [r[...][0] + r[...][1]
         for r in (agga_ref, aggb_ref, aggc_ref, aggd_ref)], axis=1)
    a = a * _inv_deg(deg_ref)
    s1 = s1_ref[...]
    h1m = jax.nn.relu(s1[:, 0:64] + a[:, 0:64])
    h1l = jax.nn.relu(s1[:, 64:128] + a[:, 64:128])
    h1t = jax.nn.relu(s1[:, 128:192] + a[:, 128:192])
    z2 = jnp.concatenate(
        [_dot(h1m, mWn2[...]), _dot(h1l, lWn2[...]), _dot(h1t, tWn2[...])],
        axis=1)
    z2a_ref[...] = z2[:, 0:48]
    z2b_ref[...] = z2[:, 48:96]
    s2_ref[...] = jnp.concatenate(
        [_dot(h1m, mWs2[...]) + mb2[...],
         _dot(h1l, lWs2[...]) + lb2[...],
         _dot(h1t, tWs2[...]) + tb2[...]],
        axis=1)


def _tc2(s1, agg1, degp, ws):
    specs = ([_rows(192)] +
             [pl.BlockSpec((2, _R, _W), lambda i: (0, i, 0))] * 4 +
             [pl.BlockSpec((2, _R, 16), lambda i: (0, i, 0))] +
             [_full(w.shape) for w in ws])
    return pl.pallas_call(
        _tc2_body,
        grid=(_G,),
        in_specs=specs,
        out_specs=[_rows(48), _rows(48), _rows(96)],
        out_shape=[jax.ShapeDtypeStruct((_N, 48), _f32),
                   jax.ShapeDtypeStruct((_N, 48), _f32),
                   jax.ShapeDtypeStruct((_N, 96), _f32)],
    )(s1, *agg1, degp, *ws)


def _tc3_body(s2_ref, agga_ref, aggb_ref, deg_ref,
              vW1, vb1, vW2, vb2, cW1, cb1, cW2, cb2,
              esm_ref, esl_ref, est_ref, root_ref, fm_ref, ty_ref):
    i = pl.program_id(0)
    a = jnp.concatenate(
        [r[...][0] + r[...][1] for r in (agga_ref, aggb_ref)], axis=1)
    a = a * _inv_deg(deg_ref)
    s2 = s2_ref[...]
    h2m = jax.nn.relu(s2[:, 0:32] + a[:, 0:32])
    h2l = jax.nn.relu(s2[:, 32:64] + a[:, 32:64])
    h2t = jax.nn.relu(s2[:, 64:96] + a[:, 64:96])
    esm_ref[...] = h2m
    esl_ref[...] = h2l
    est_ref[...] = h2t
    h2 = jnp.concatenate([h2m, h2l, h2t], axis=1)
    root_ref[...] = _dot(jax.nn.relu(_dot(h2, vW1[...]) + vb1[...]), vW2[...]) + vb2[...]

    @pl.when(i == 0)
    def _():
        fm_ref[...] = jnp.zeros((1, 96), _f32)

    fm_ref[...] += jnp.sum(h2, axis=0, keepdims=True)

    @pl.when(i == _G - 1)
    def _():
        f = fm_ref[...] * (1.0 / _N)
        fm_ref[...] = f
        ty_ref[...] = _dot(jax.nn.relu(_dot(f, cW1[...]) + cb1[...]), cW2[...]) + cb2[...]


def _tc3(s2, agg2, degp, ws):
    specs = ([_rows(96)] +
             [pl.BlockSpec((2, _R, _W), lambda i: (0, i, 0))] * 2 +
             [pl.BlockSpec((2, _R, 16), lambda i: (0, i, 0))] +
             [_full(w.shape) for w in ws])
    return pl.pallas_call(
        _tc3_body,
        grid=(_G,),
        in_specs=specs,
        out_specs=[_rows(32), _rows(32), _rows(32), _rows(1),
                   pl.BlockSpec((1, 96), lambda i: (0, 0)),
                   pl.BlockSpec((1, 5), lambda i: (0, 0))],
        out_shape=[jax.ShapeDtypeStruct((_N, 32), _f32),
                   jax.ShapeDtypeStruct((_N, 32), _f32),
                   jax.ShapeDtypeStruct((_N, 32), _f32),
                   jax.ShapeDtypeStruct((_N, 1), _f32),
                   jax.ShapeDtypeStruct((1, 96), _f32),
                   jax.ShapeDtypeStruct((1, 5), _f32)],
    )(s2, *agg2, degp, *ws)


def kernel(metric, log, trace, edge_index, params):
    p = params
    xm = metric.reshape(_N, 240)
    xt = trace.reshape(_N, 80)

    src = edge_index[0]
    dst = edge_index[1]
    npad = _EPAD - _E
    srcp = jnp.concatenate([src, jnp.zeros((npad,), jnp.int32)]).reshape(-1, _CHUNK)
    dstp = jnp.concatenate([dst, jnp.full((npad,), _PAD_DST, jnp.int32)]).reshape(-1, _CHUNK)

    zeros48 = jnp.zeros((_NPAD, _W), _f32)
    zeros16 = jnp.zeros((_NPAD, 16), _f32)
    ones16 = jnp.ones((_CHUNK, 16), _f32)

    mW36 = p['m_conv_W'].transpose(2, 1, 0).reshape(36, 64)
    tW12 = p['t_conv_W'].transpose(2, 1, 0).reshape(12, 64)

    degp = _sc_degree(dstp, zeros16, ones16)

    tc1_ws = [mW36, p['m_conv_b'], p['m_proj_W'], p['m_proj_b'],
              p['l_W1'], p['l_b1'], p['l_W2'], p['l_b2'],
              tW12, p['t_conv_b'], p['t_proj_W'], p['t_proj_b'],
              p['metric_Wn1'], p['metric_Ws1'], p['metric_b1'],
              p['log_Wn1'], p['log_Ws1'], p['log_b1'],
              p['trace_Wn1'], p['trace_Ws1'], p['trace_b1']]
    z1a, z1b, z1c, z1d, s1 = _tc1(xm, log, xt, tc1_ws)

    agg1 = _agg([z1a, z1b, z1c, z1d], srcp, dstp, zeros48)

    tc2_ws = [p['metric_Wn2'], p['metric_Ws2'], p['metric_b2'],
              p['log_Wn2'], p['log_Ws2'], p['log_b2'],
              p['trace_Wn2'], p['trace_Ws2'], p['trace_b2']]
    z2a, z2b, s2 = _tc2(s1, agg1, degp, tc2_ws)

    agg2 = _agg([z2a, z2b], srcp, dstp, zeros48)

    tc3_ws = [p['vot_W1'], p['vot_b1'], p['vot_W2'], p['vot_b2'],
              p['cls_W1'], p['cls_b1'], p['cls_W2'], p['cls_b2']]
    esm, esl, est, root, fmean, typ = _tc3(s2, agg2, degp, tc3_ws)

    return (fmean[:, 0:32], fmean[:, 32:64], fmean[:, 64:96],
            esm, esl, est, root, typ)


# speedup vs baseline: 1.0281x; 1.0281x over previous
import functools

import jax
import jax.numpy as jnp
from jax import lax
from jax.experimental import pallas as pl
from jax.experimental.pallas import tpu as pltpu
from jax.experimental.pallas import tpu_sc as plsc

_N = 10000
_E = 320000
_NPAD = 10240
_EPAD = 327680
_CHUNK = 128
_TILES = 32
_PER_TILE = _EPAD // _TILES
_NCHUNKS = _PER_TILE // _CHUNK
_RPT = _NPAD // 16
_PAD_DST = 10016

_R = 1000
_G = _N // _R

_f32 = jnp.float32

@functools.cache
def _sc_mesh():
    return plsc.VectorSubcoreMesh(
        core_axis_name="c", subcore_axis_name="s", num_cores=2, num_subcores=16)


_sc_params = pltpu.CompilerParams(use_tc_tiling_on_sc=False)


_NB = 5


_W = 48


_W = 48


def _agg(zs, src2d, dst2d, zeros):
    nq = len(zs)

    @functools.partial(
        pl.kernel,
        out_type=[jax.ShapeDtypeStruct((2, _NPAD, _W), _f32)] * nq,
        mesh=_sc_mesh(),
        scratch_types=[
            pltpu.VMEM((_NCHUNKS, _CHUNK), jnp.int32),
            pltpu.VMEM((_NCHUNKS, _CHUNK), jnp.int32),
            pltpu.VMEM((_NB, _CHUNK, _W), _f32),
            pltpu.VMEM_SHARED((_NPAD, _W), _f32),
            pltpu.VMEM_SHARED((_N, _W), _f32),
            pltpu.SemaphoreType.DMA,
            pltpu.SemaphoreType.DMA,
        ],
        compiler_params=_sc_params,
    )
    def k(*refs):
        z_hbms = refs[:nq]
        src_hbm, dst_hbm, zero_hbm = refs[nq:nq + 3]
        out_hbms = refs[nq + 3:2 * nq + 3]
        srci, dsti, rows, acc, zbuf, gsem, ssem = refs[2 * nq + 3:]
        cid = lax.axis_index("c")
        sid = lax.axis_index("s")
        tile = cid * 16 + sid
        rbase = sid * _RPT
        cbase = tile * _NCHUNKS
        pltpu.sync_copy(src_hbm.at[pl.ds(cbase, _NCHUNKS)], srci)
        pltpu.sync_copy(dst_hbm.at[pl.ds(cbase, _NCHUNKS)], dsti)

        for q in range(nq):
            pltpu.sync_copy(zero_hbm.at[pl.ds(rbase, _RPT)],
                            acc.at[pl.ds(rbase, _RPT)])
            pltpu.sync_copy(z_hbms[q].at[pl.ds(sid * 625, 625)],
                            zbuf.at[pl.ds(sid * 625, 625)])
            plsc.subcore_barrier()

            for b in range(_NB):
                pltpu.async_copy(zbuf.at[srci.at[b]], rows.at[b], gsem)

            @pl.loop(0, _NCHUNKS // _NB)
            def _(s):
                for b in range(_NB):
                    j = s * _NB + b
                    pltpu.make_async_copy(
                        zbuf.at[srci.at[j]], rows.at[b], gsem).wait()
                    pltpu.async_copy(rows.at[b], acc.at[dsti.at[j]], ssem,
                                     add=True)
                for b in range(_NB):
                    j = s * _NB + b
                    pltpu.make_async_copy(
                        rows.at[b], acc.at[dsti.at[j]], ssem).wait()

                    @pl.when(j + _NB < _NCHUNKS)
                    def _():
                        pltpu.async_copy(zbuf.at[srci.at[j + _NB]],
                                         rows.at[b], gsem)

            plsc.subcore_barrier()
            pltpu.sync_copy(acc.at[pl.ds(rbase, _RPT)],
                            out_hbms[q].at[cid, pl.ds(rbase, _RPT)])

    return k(*zs, src2d, dst2d, zeros)


def _sc_degree(dstp, zeros16, ones16):
    @functools.partial(
        pl.kernel,
        out_type=jax.ShapeDtypeStruct((2, _NPAD, 16), _f32),
        mesh=_sc_mesh(),
        scratch_types=[
            pltpu.VMEM((_NCHUNKS, _CHUNK), jnp.int32),
            pltpu.VMEM((_CHUNK, 16), _f32),
            pltpu.VMEM_SHARED((_NPAD, 16), _f32),
        ],
        compiler_params=_sc_params,
    )
    def k(dst_hbm, zero_hbm, ones_hbm, out_hbm, dsti, ones_v, acc):
        cid = lax.axis_index("c")
        sid = lax.axis_index("s")
        tile = cid * 16 + sid
        rbase = sid * _RPT
        cbase = tile * _NCHUNKS
        pltpu.sync_copy(zero_hbm.at[pl.ds(rbase, _RPT)], acc.at[pl.ds(rbase, _RPT)])
        pltpu.sync_copy(dst_hbm.at[pl.ds(cbase, _NCHUNKS)], dsti)
        pltpu.sync_copy(ones_hbm, ones_v)
        plsc.subcore_barrier()

        @pl.loop(0, _NCHUNKS)
        def _(j):
            pltpu.sync_copy(ones_v, acc.at[dsti.at[j]], add=True)

        plsc.subcore_barrier()
        pltpu.sync_copy(acc.at[pl.ds(rbase, _RPT)],
                        out_hbm.at[cid, pl.ds(rbase, _RPT)])

    return k(dstp, zeros16, ones16)


def _dot(a, b):
    return jnp.dot(a, b, preferred_element_type=_f32)


def _tc1_body(xm_ref, lg_ref, xt_ref,
              mW36, mcb, mpW, mpb,
              lW1, lEb1, lW2, lEb2,
              tW12, tcb, tpW, tpb,
              mWn1, mWs1, mb1, lWn1, lWs1, lb1, tWn1, tWs1, tb1,
              z1a_ref, z1b_ref, z1c_ref, z1d_ref, s1_ref):
    xm = xm_ref[...]
    accm = jnp.zeros((_R, 64), _f32)
    for t in range(18):
        accm = accm + jax.nn.relu(_dot(xm[:, 12 * t:12 * t + 36], mW36[...]) + mcb[...])
    m_emb = _dot(accm * (1.0 / 18.0), mpW[...]) + mpb[...]

    lg = lg_ref[...]
    l_emb = _dot(jax.nn.relu(_dot(lg, lW1[...]) + lEb1[...]), lW2[...]) + lEb2[...]

    xt = xt_ref[...]
    acct = jnp.zeros((_R, 64), _f32)
    for t in range(18):
        acct = acct + jax.nn.relu(_dot(xt[:, 4 * t:4 * t + 12], tW12[...]) + tcb[...])
    t_emb = _dot(acct * (1.0 / 18.0), tpW[...]) + tpb[...]

    z1 = jnp.concatenate(
        [_dot(m_emb, mWn1[...]), _dot(l_emb, lWn1[...]), _dot(t_emb, tWn1[...])],
        axis=1)
    z1a_ref[...] = z1[:, 0:48]
    z1b_ref[...] = z1[:, 48:96]
    z1c_ref[...] = z1[:, 96:144]
    z1d_ref[...] = z1[:, 144:192]
    s1_ref[...] = jnp.concatenate(
        [_dot(m_emb, mWs1[...]) + mb1[...],
         _dot(l_emb, lWs1[...]) + lb1[...],
         _dot(t_emb, tWs1[...]) + tb1[...]],
        axis=1)


def _full(shape):
    return pl.BlockSpec(shape, lambda i: tuple(0 for _ in shape))


def _rows(width):
    return pl.BlockSpec((_R, width), lambda i: (i, 0))


def _tc1(xm, lg, xt, ws):
    specs = ([_rows(240), _rows(100), _rows(80)] +
             [_full(w.shape) for w in ws])
    return pl.pallas_call(
        _tc1_body,
        grid=(_G,),
        in_specs=specs,
        out_specs=[_rows(48), _rows(48), _rows(48), _rows(48), _rows(192)],
        out_shape=[jax.ShapeDtypeStruct((_N, 48), _f32)] * 4 +
                  [jax.ShapeDtypeStruct((_N, 192), _f32)],
    )(xm, lg, xt, *ws)


def _inv_deg(deg_ref):
    d = deg_ref[...]
    dg = d[0, :, 0:1] + d[1, :, 0:1]
    return 1.0 / jnp.maximum(dg, 1.0)


def _tc2_body(s1_ref, agga_ref, aggb_ref, aggc_ref, aggd_ref, deg_ref,
              mWn2, mWs2, mb2, lWn2, lWs2, lb2, tWn2, tWs2, tb2,
              z2a_ref, z2b_ref, s2_ref):
    a = jnp.concatenate(
        [r[...][0] + r[...][1]
         for r in (agga_ref, aggb_ref, aggc_ref, aggd_ref)], axis=1)
    a = a * _inv_deg(deg_ref)
    s1 = s1_ref[...]
    h1m = jax.nn.relu(s1[:, 0:64] + a[:, 0:64])
    h1l = jax.nn.relu(s1[:, 64:128] + a[:, 64:128])
    h1t = jax.nn.relu(s1[:, 128:192] + a[:, 128:192])
    z2 = jnp.concatenate(
        [_dot(h1m, mWn2[...]), _dot(h1l, lWn2[...]), _dot(h1t, tWn2[...])],
        axis=1)
    z2a_ref[...] = z2[:, 0:48]
    z2b_ref[...] = z2[:, 48:96]
    s2_ref[...] = jnp.concatenate(
        [_dot(h1m, mWs2[...]) + mb2[...],
         _dot(h1l, lWs2[...]) + lb2[...],
         _dot(h1t, tWs2[...]) + tb2[...]],
        axis=1)


def _tc2(s1, agg1, degp, ws):
    specs = ([_rows(192)] +
             [pl.BlockSpec((2, _R, _W), lambda i: (0, i, 0))] * 4 +
             [pl.BlockSpec((2, _R, 16), lambda i: (0, i, 0))] +
             [_full(w.shape) for w in ws])
    return pl.pallas_call(
        _tc2_body,
        grid=(_G,),
        in_specs=specs,
        out_specs=[_rows(48), _rows(48), _rows(96)],
        out_shape=[jax.ShapeDtypeStruct((_N, 48), _f32),
                   jax.ShapeDtypeStruct((_N, 48), _f32),
                   jax.ShapeDtypeStruct((_N, 96), _f32)],
    )(s1, *agg1, degp, *ws)


def _tc3_body(s2_ref, agga_ref, aggb_ref, deg_ref,
              vW1, vb1, vW2, vb2, cW1, cb1, cW2, cb2,
              esm_ref, esl_ref, est_ref, root_ref, fm_ref, ty_ref):
    i = pl.program_id(0)
    a = jnp.concatenate(
        [r[...][0] + r[...][1] for r in (agga_ref, aggb_ref)], axis=1)
    a = a * _inv_deg(deg_ref)
    s2 = s2_ref[...]
    h2m = jax.nn.relu(s2[:, 0:32] + a[:, 0:32])
    h2l = jax.nn.relu(s2[:, 32:64] + a[:, 32:64])
    h2t = jax.nn.relu(s2[:, 64:96] + a[:, 64:96])
    esm_ref[...] = h2m
    esl_ref[...] = h2l
    est_ref[...] = h2t
    h2 = jnp.concatenate([h2m, h2l, h2t], axis=1)
    root_ref[...] = _dot(jax.nn.relu(_dot(h2, vW1[...]) + vb1[...]), vW2[...]) + vb2[...]

    @pl.when(i == 0)
    def _():
        fm_ref[...] = jnp.zeros((1, 96), _f32)

    fm_ref[...] += jnp.sum(h2, axis=0, keepdims=True)

    @pl.when(i == _G - 1)
    def _():
        f = fm_ref[...] * (1.0 / _N)
        fm_ref[...] = f
        ty_ref[...] = _dot(jax.nn.relu(_dot(f, cW1[...]) + cb1[...]), cW2[...]) + cb2[...]


def _tc3(s2, agg2, degp, ws):
    specs = ([_rows(96)] +
             [pl.BlockSpec((2, _R, _W), lambda i: (0, i, 0))] * 2 +
             [pl.BlockSpec((2, _R, 16), lambda i: (0, i, 0))] +
             [_full(w.shape) for w in ws])
    return pl.pallas_call(
        _tc3_body,
        grid=(_G,),
        in_specs=specs,
        out_specs=[_rows(32), _rows(32), _rows(32), _rows(1),
                   pl.BlockSpec((1, 96), lambda i: (0, 0)),
                   pl.BlockSpec((1, 5), lambda i: (0, 0))],
        out_shape=[jax.ShapeDtypeStruct((_N, 32), _f32),
                   jax.ShapeDtypeStruct((_N, 32), _f32),
                   jax.ShapeDtypeStruct((_N, 32), _f32),
                   jax.ShapeDtypeStruct((_N, 1), _f32),
                   jax.ShapeDtypeStruct((1, 96), _f32),
                   jax.ShapeDtypeStruct((1, 5), _f32)],
    )(s2, *agg2, degp, *ws)


def kernel(metric, log, trace, edge_index, params):
    p = params
    xm = metric.reshape(_N, 240)
    xt = trace.reshape(_N, 80)

    src = edge_index[0]
    dst = edge_index[1]
    npad = _EPAD - _E
    srcp = jnp.concatenate([src, jnp.zeros((npad,), jnp.int32)]).reshape(-1, _CHUNK)
    dstp = jnp.concatenate([dst, jnp.full((npad,), _PAD_DST, jnp.int32)]).reshape(-1, _CHUNK)

    zeros48 = jnp.zeros((_NPAD, _W), _f32)
    zeros16 = jnp.zeros((_NPAD, 16), _f32)
    ones16 = jnp.ones((_CHUNK, 16), _f32)

    mW36 = p['m_conv_W'].transpose(2, 1, 0).reshape(36, 64)
    tW12 = p['t_conv_W'].transpose(2, 1, 0).reshape(12, 64)

    degp = _sc_degree(dstp, zeros16, ones16)

    tc1_ws = [mW36, p['m_conv_b'], p['m_proj_W'], p['m_proj_b'],
              p['l_W1'], p['l_b1'], p['l_W2'], p['l_b2'],
              tW12, p['t_conv_b'], p['t_proj_W'], p['t_proj_b'],
              p['metric_Wn1'], p['metric_Ws1'], p['metric_b1'],
              p['log_Wn1'], p['log_Ws1'], p['log_b1'],
              p['trace_Wn1'], p['trace_Ws1'], p['trace_b1']]
    z1a, z1b, z1c, z1d, s1 = _tc1(xm, log, xt, tc1_ws)

    agg1 = [_agg([z], srcp, dstp, zeros48)[0] for z in (z1a, z1b, z1c, z1d)]

    tc2_ws = [p['metric_Wn2'], p['metric_Ws2'], p['metric_b2'],
              p['log_Wn2'], p['log_Ws2'], p['log_b2'],
              p['trace_Wn2'], p['trace_Ws2'], p['trace_b2']]
    z2a, z2b, s2 = _tc2(s1, agg1, degp, tc2_ws)

    agg2 = [_agg([z], srcp, dstp, zeros48)[0] for z in (z2a, z2b)]

    tc3_ws = [p['vot_W1'], p['vot_b1'], p['vot_W2'], p['vot_b2'],
              p['cls_W1'], p['cls_b1'], p['cls_W2'], p['cls_b2']]
    esm, esl, est, root, fmean, typ = _tc3(s2, agg2, degp, tc3_ws)

    return (fmean[:, 0:32], fmean[:, 32:64], fmean[:, 64:96],
            esm, esl, est, root, typ)
